# baseline (speedup 1.0000x reference)
import functools

import jax
import jax.numpy as jnp
from jax import lax
from jax.experimental import pallas as pl
from jax.experimental.pallas import tpu as pltpu
from jax.experimental.pallas import tpu_sc as plsc

_N_NODES = 10000
_N_GRAPHS = 64
_DIM = 128
_IR0, _IR1 = 0.5, 5.0
_DEV_COEFF = 0.2
_ES0, _ES1 = 0.0178, 0.0356
_N_SHORT, _N_LONG = 10.0, 6.0
_HB = (-0.7, 0.0)
_ML = (-0.7, 0.0)
_HP = (0.5, 1.5)


_NW = 32
_CH = 448


def _sc_gather_multi(pairs, e, out_dims, ch=None):
    ch = ch or _CH
    k_chunks = -(-e // (_NW * ch))
    k_chunks += k_chunks % 2
    e_pad = _NW * ch * k_chunks
    per_w = ch * k_chunks
    mesh = plsc.VectorSubcoreMesh(core_axis_name="c", subcore_axis_name="s")
    n = len(pairs)
    d = out_dims[0]
    assert all(x == d for x in out_dims)

    idx_arrays = []
    idx_slot = []
    for _, ia in pairs:
        for s, a in enumerate(idx_arrays):
            if a is ia:
                idx_slot.append(s)
                break
        else:
            idx_slot.append(len(idx_arrays))
            idx_arrays.append(ia)
    idx_arrays = [jnp.pad(a.astype(jnp.int32), (0, e_pad - e))
                  for a in idx_arrays]
    ni = len(idx_arrays)
    total = k_chunks * n

    @functools.partial(
        pl.kernel,
        mesh=mesh,
        out_type=[jax.ShapeDtypeStruct((e_pad, d) if d else (e_pad,),
                                       jnp.float32)
                  for _ in range(n)],
        scratch_types=[pltpu.VMEM((per_w,), jnp.int32) for _ in range(ni)]
                      + [pltpu.VMEM((ch, d) if d else (ch,), jnp.float32)
                         for _ in range(2)]
                      + [pltpu.SemaphoreType.DMA, pltpu.SemaphoreType.DMA],
    )
    def _k(*refs):
        tables = refs[0:n]
        idxs_hbm = refs[n:n + ni]
        outs = refs[n + ni:2 * n + ni]
        idx_v = refs[2 * n + ni:2 * n + 2 * ni]
        ring = refs[2 * n + 2 * ni:2 * n + 2 * ni + 2]
        sem_g, sem_w = refs[-2:]
        wid = lax.axis_index("s") * 2 + lax.axis_index("c")
        base = wid * per_w
        for s in range(ni):
            pltpu.sync_copy(idxs_hbm[s].at[pl.ds(base, per_w)], idx_v[s])

        njobs = 2 * n

        def body(ci, carry):
            def issue(q):
                c = ci * 2 + q // n
                j = q % n
                isl = idx_v[idx_slot[j]].at[pl.ds(c * ch, ch)]
                return pltpu.async_copy(tables[j].at[isl], ring[q % 2], sem_g)

            gq = issue(0)
            wprev = None
            for q in range(njobs):
                gq.wait()
                if wprev is not None:
                    wprev.wait()
                if q + 1 < njobs:
                    gq = issue(q + 1)
                c = ci * 2 + q // n
                j = q % n
                wprev = pltpu.async_copy(
                    ring[q % 2], outs[j].at[pl.ds(base + c * ch, ch)], sem_w)
            wprev.wait()
            return carry

        lax.fori_loop(0, k_chunks // 2, body, 0)

    flat = [t for t, _ in pairs] + idx_arrays
    return _k(*flat)


def _mm_body(x_ref, w_ref, b_ref, o_ref, *, act):
    y = jnp.dot(x_ref[...], w_ref[...], preferred_element_type=jnp.float32)
    y = y + b_ref[...][None, :]
    if act == "relu":
        y = jnp.maximum(y, 0.0)
    o_ref[...] = y


def _mm(x, w, b=None, act=None, bn=2000):
    n, k = x.shape
    f = w.shape[1]
    if b is None:
        b = jnp.zeros((f,), jnp.float32)
    return pl.pallas_call(
        functools.partial(_mm_body, act=act),
        grid=(n // bn,),
        in_specs=[
            pl.BlockSpec((bn, k), lambda i: (i, 0)),
            pl.BlockSpec((k, f), lambda i: (0, 0)),
            pl.BlockSpec((f,), lambda i: (0,)),
        ],
        out_specs=pl.BlockSpec((bn, f), lambda i: (i, 0)),
        out_shape=jax.ShapeDtypeStruct((n, f), jnp.float32),
    )(x, w, b)


def _gate_body(x_ref, m_ref, wt_ref, wb_ref, gb_ref, den_ref, o_ref, *, relu_m,
               use_den):
    m = m_ref[...]
    if use_den:
        m = m / (den_ref[...] + 1e-16)
    if relu_m:
        m = jnp.maximum(m, 0.0)
    logit = (
        jnp.dot(x_ref[...], wt_ref[...], preferred_element_type=jnp.float32)
        + jnp.dot(m, wb_ref[...], preferred_element_type=jnp.float32)
        + gb_ref[0]
    )
    c = jax.nn.sigmoid(logit)
    o_ref[...] = c * x_ref[...] + (1.0 - c) * m


def _gate(x, m, gw, gb, relu_m, denom=None, bn=2000):
    n, k = x.shape
    wt, wb = gw[:k], gw[k:]
    use_den = denom is not None
    if denom is None:
        denom = jnp.ones((n, 1), jnp.float32)
    return pl.pallas_call(
        functools.partial(_gate_body, relu_m=relu_m, use_den=use_den),
        grid=(n // bn,),
        in_specs=[
            pl.BlockSpec((bn, k), lambda i: (i, 0)),
            pl.BlockSpec((bn, k), lambda i: (i, 0)),
            pl.BlockSpec((k, 1), lambda i: (0, 0)),
            pl.BlockSpec((k, 1), lambda i: (0, 0)),
            pl.BlockSpec((1,), lambda i: (0,)),
            pl.BlockSpec((bn, 1), lambda i: (i, 0)),
        ],
        out_specs=pl.BlockSpec((bn, k), lambda i: (i, 0)),
        out_shape=jax.ShapeDtypeStruct((n, k), jnp.float32),
    )(x, m, wt, wb, gb, denom)


def _pair_body(ga_ref, gb_ref,
               b1e_ref, w2e_ref, b2e_ref, b1d_ref, w2d_ref, b2d_ref,
               rotor_ref, cf_ref, en_ref, dv_ref):
    i = pl.program_id(0)
    nsteps = pl.num_programs(0)

    ga = ga_ref[...]
    gb = gb_ref[...]
    ae, ad, s0 = ga[:, :_DIM], ga[:, _DIM:2 * _DIM], ga[:, 2 * _DIM:]
    be, bd, s1 = gb[:, :_DIM], gb[:, _DIM:2 * _DIM], gb[:, 2 * _DIM:]

    he = jnp.maximum(ae + be + b1e_ref[...][None, :], 0.0)
    eps_logit = jnp.dot(he, w2e_ref[...], preferred_element_type=jnp.float32) + b2e_ref[0]
    eps = jax.nn.sigmoid(eps_logit[:, 0]) * (_ES1 - _ES0) + _ES0

    hd = jnp.maximum(ad + bd + b1d_ref[...][None, :], 0.0)
    dv_logit = jnp.dot(hd, w2d_ref[...], preferred_element_type=jnp.float32) + b2d_ref[0]
    dvdw = jnp.tanh(dv_logit[:, 0]) * _DEV_COEFF
    dx = s0[:, 0] - s1[:, 0]
    dy = s0[:, 1] - s1[:, 1]
    dz = s0[:, 2] - s1[:, 2]
    D = jnp.sqrt(dx * dx + dy * dy + dz * dz + 1e-12)

    lig0, lig1 = s0[:, 4], s1[:, 4]
    met0, met1 = s0[:, 5], s1[:, 5]
    don0, don1 = s0[:, 6], s1[:, 6]
    acc0, acc1 = s0[:, 7], s1[:, 7]
    hyd0, hyd1 = s0[:, 8], s1[:, 8]
    bat0, bat1 = s0[:, 9], s1[:, 9]

    pair_ok = lig0 * (1.0 - lig1) * (bat0 == bat1).astype(jnp.float32)
    maskf = pair_ok * (D >= _IR0).astype(jnp.float32) * (D <= _IR1).astype(jnp.float32)

    R = s0[:, 3] + s1[:, 3] + dvdw
    Dc = jnp.maximum(D, _IR0)
    ratio = R / Dc
    lj = jnp.minimum(ratio ** _N_SHORT - 2.0 * ratio ** _N_LONG, 100.0) * eps

    hbc = cf_ref[0]
    hpc = cf_ref[1]
    rc = cf_ref[2]
    min_hb = -(hbc * hbc)
    min_hp = -(hpc * hpc)
    dev = Dc - R

    def _lp(minima, c0, c1):
        frac = jnp.clip((c1 - dev) / (c1 - c0), 0.0, 1.0)
        return minima * frac

    e_hb = _lp(min_hb, _HB[0], _HB[1])
    e_ml = _lp(min_hb, _ML[0], _ML[1])
    e_hp = _lp(min_hp, _HP[0], _HP[1])

    not_metal = (1.0 - met0) * (1.0 - met1)
    m_hb = jnp.minimum(don0 * acc1 + acc0 * don1, 1.0) * not_metal
    m_ml = jnp.minimum(met0 * acc1 + acc0 * met1, 1.0)
    m_hp = hyd0 * hyd1 * not_metal

    ep = jnp.stack(
        [lj * not_metal, e_hb * m_hb, e_ml * m_ml, e_hp * m_hp], axis=1
    ) * maskf[:, None]

    gi = jax.lax.broadcasted_iota(jnp.int32, (_N_GRAPHS, ep.shape[0]), 0)
    onehot = (gi == bat0.astype(jnp.int32)[None, :]).astype(jnp.float32)
    part = jnp.dot(onehot, ep, preferred_element_type=jnp.float32,
                   precision=jax.lax.Precision.HIGHEST)

    @pl.when(i == 0)
    def _():
        en_ref[...] = jnp.zeros_like(en_ref)

    en_ref[...] += part

    @pl.when(i == nsteps - 1)
    def _():
        penalty = 1.0 + rc * rc * rotor_ref[...]
        en_ref[...] = en_ref[...] / penalty

    dv_ref[...] = (dvdw * maskf)[None, None, :]


def _pair_stage(ga, gb, pe, pd, rotor, coeffs, eb=2000):
    e = ga.shape[0]
    grid = e // eb
    en, dv = pl.pallas_call(
        _pair_body,
        grid=(grid,),
        in_specs=[
            pl.BlockSpec((eb, 3 * _DIM), lambda i: (i, 0)),
            pl.BlockSpec((eb, 3 * _DIM), lambda i: (i, 0)),
            pl.BlockSpec((_DIM,), lambda i: (0,)),
            pl.BlockSpec((_DIM, 1), lambda i: (0, 0)),
            pl.BlockSpec((1,), lambda i: (0,)),
            pl.BlockSpec((_DIM,), lambda i: (0,)),
            pl.BlockSpec((_DIM, 1), lambda i: (0, 0)),
            pl.BlockSpec((1,), lambda i: (0,)),
            pl.BlockSpec((_N_GRAPHS, 1), lambda i: (0, 0)),
            pl.BlockSpec((3,), lambda i: (0,)),
        ],
        out_specs=[
            pl.BlockSpec((_N_GRAPHS, 4), lambda i: (0, 0)),
            pl.BlockSpec((1, 1, eb), lambda i: (i, 0, 0)),
        ],
        out_shape=[
            jax.ShapeDtypeStruct((_N_GRAPHS, 4), jnp.float32),
            jax.ShapeDtypeStruct((grid, 1, eb), jnp.float32),
        ],
    )(ga, gb, pe["b1"], pe["W2"], pe["b2"],
      pd["b1"], pd["W2"], pd["b2"], rotor, coeffs)
    return en, dv.reshape(e)


def kernel(x, edge_index, edge_index_c, edge_index_i, pos, vdw_radii, batch,
           is_ligand, is_metal, is_h_donor, is_h_acceptor, is_hydrophobic,
           rotor, params):
    h = _mm(x, params["embed_W"])

    src, dst = edge_index[0], edge_index[1]
    e_intra = src.shape[0]
    eye = jnp.eye(_DIM, dtype=jnp.float32)
    for p in params["gat"]:
        h1 = _mm(h, p["W"], p["b"])
        pcat = _mm(h1, jnp.concatenate([eye, p["A"]], axis=1))
        ps, pd = _sc_gather_multi([(pcat, src), (pcat, dst)], e_intra,
                                  [2 * _DIM, 2 * _DIM], ch=192)
        ps, pd = ps[:e_intra], pd[:e_intra]
        e = jnp.sum(ps[:, _DIM:] * pd[:, :_DIM]
                    + pd[:, _DIM:] * ps[:, :_DIM], -1)
        w = jnp.exp(jnp.minimum(e, 80.0))
        cat = jnp.concatenate([w[:, None] * ps[:, :_DIM], w[:, None]], axis=1)
        s = jax.ops.segment_sum(cat, dst, num_segments=_N_NODES)
        h = _gate(h, s[:, :_DIM], p["gW"], p["gb"], relu_m=True,
                  denom=s[:, _DIM:])

    srcc, dstc = edge_index_c[0], edge_index_c[1]
    e_inter = srcc.shape[0]
    for p in params["inter"]:
        hr = _mm(h, p["W"], p["b"], act="relu")
        (ghr,) = _sc_gather_multi([(hr, srcc)], e_inter, [_DIM])
        m = jax.ops.segment_sum(ghr[:e_inter], dstc, num_segments=_N_NODES)
        h = _gate(h, m, p["gW"], p["gb"], relu_m=False)

    pe, pdv = params["vdw_eps"], params["dvdw"]

    scal = jnp.concatenate(
        [
            pos,
            vdw_radii[:, None],
            is_ligand[:, None].astype(jnp.float32),
            is_metal[:, None].astype(jnp.float32),
            is_h_donor[:, None].astype(jnp.float32),
            is_h_acceptor[:, None].astype(jnp.float32),
            is_hydrophobic[:, None].astype(jnp.float32),
            batch[:, None].astype(jnp.float32),
            jnp.zeros((_N_NODES, _DIM - 10), jnp.float32),
        ],
        axis=1,
    )

    i0, i1 = edge_index_i[0], edge_index_i[1]
    e_pair = i0.shape[0]
    t0 = jnp.concatenate(
        [_mm(h, jnp.concatenate([pe["W1"][:_DIM], pdv["W1"][:_DIM]], axis=1)),
         scal], axis=1)
    t1 = jnp.concatenate(
        [_mm(h, jnp.concatenate([pe["W1"][_DIM:], pdv["W1"][_DIM:]], axis=1)),
         scal], axis=1)
    ga, gb = _sc_gather_multi([(t0, i0), (t1, i1)], e_pair,
                              [3 * _DIM, 3 * _DIM], ch=128)
    ga, gb = ga[:e_pair], gb[:e_pair]

    coeffs = jnp.concatenate(
        [params["hbond_coeff"], params["hydrophobic_coeff"], params["rotor_coeff"]]
    )
    energies, dvdw_masked = _pair_stage(ga, gb, pe, pdv, rotor, coeffs)
    return energies, dvdw_masked


# speedup vs baseline: 1.1022x; 1.1022x over previous
import functools

import jax
import jax.numpy as jnp
from jax import lax
from jax.experimental import pallas as pl
from jax.experimental.pallas import tpu as pltpu
from jax.experimental.pallas import tpu_sc as plsc

_N_NODES = 10000
_N_GRAPHS = 64
_DIM = 128
_IR0, _IR1 = 0.5, 5.0
_DEV_COEFF = 0.2
_ES0, _ES1 = 0.0178, 0.0356
_N_SHORT, _N_LONG = 10.0, 6.0
_HB = (-0.7, 0.0)
_ML = (-0.7, 0.0)
_HP = (0.5, 1.5)


_NW = 32
_CH = 448


def _sc_gather_multi(pairs, e, out_dims, ch=None):
    ch = ch or _CH
    k_chunks = -(-e // (_NW * ch))
    k_chunks += k_chunks % 2
    e_pad = _NW * ch * k_chunks
    per_w = ch * k_chunks
    mesh = plsc.VectorSubcoreMesh(core_axis_name="c", subcore_axis_name="s")
    n = len(pairs)
    d = out_dims[0]
    assert all(x == d for x in out_dims)

    idx_arrays = []
    idx_slot = []
    for _, ia in pairs:
        for s, a in enumerate(idx_arrays):
            if a is ia:
                idx_slot.append(s)
                break
        else:
            idx_slot.append(len(idx_arrays))
            idx_arrays.append(ia)
    idx_arrays = [jnp.pad(a.astype(jnp.int32), (0, e_pad - e))
                  for a in idx_arrays]
    ni = len(idx_arrays)
    total = k_chunks * n

    @functools.partial(
        pl.kernel,
        mesh=mesh,
        out_type=[jax.ShapeDtypeStruct((e_pad, d) if d else (e_pad,),
                                       jnp.float32)
                  for _ in range(n)],
        scratch_types=[pltpu.VMEM((per_w,), jnp.int32) for _ in range(ni)]
                      + [pltpu.VMEM((ch, d) if d else (ch,), jnp.float32)
                         for _ in range(2)]
                      + [pltpu.SemaphoreType.DMA, pltpu.SemaphoreType.DMA],
    )
    def _k(*refs):
        tables = refs[0:n]
        idxs_hbm = refs[n:n + ni]
        outs = refs[n + ni:2 * n + ni]
        idx_v = refs[2 * n + ni:2 * n + 2 * ni]
        ring = refs[2 * n + 2 * ni:2 * n + 2 * ni + 2]
        sem_g, sem_w = refs[-2:]
        wid = lax.axis_index("s") * 2 + lax.axis_index("c")
        base = wid * per_w
        for s in range(ni):
            pltpu.sync_copy(idxs_hbm[s].at[pl.ds(base, per_w)], idx_v[s])

        if total <= 64:
            def issue(q):
                c, j = divmod(q, n)
                isl = idx_v[idx_slot[j]].at[pl.ds(c * ch, ch)]
                return pltpu.async_copy(tables[j].at[isl], ring[q % 2], sem_g)

            gq = issue(0)
            wprev = None
            for q in range(total):
                gq.wait()
                if wprev is not None:
                    wprev.wait()
                if q + 1 < total:
                    gq = issue(q + 1)
                c, j = divmod(q, n)
                wprev = pltpu.async_copy(
                    ring[q % 2], outs[j].at[pl.ds(base + c * ch, ch)], sem_w)
            wprev.wait()
        else:
            njobs = 2 * n

            def body(ci, carry):
                def issue(q):
                    c = ci * 2 + q // n
                    j = q % n
                    isl = idx_v[idx_slot[j]].at[pl.ds(c * ch, ch)]
                    return pltpu.async_copy(tables[j].at[isl], ring[q % 2],
                                            sem_g)

                gq = issue(0)
                wprev = None
                for q in range(njobs):
                    gq.wait()
                    if wprev is not None:
                        wprev.wait()
                    if q + 1 < njobs:
                        gq = issue(q + 1)
                    c = ci * 2 + q // n
                    j = q % n
                    wprev = pltpu.async_copy(
                        ring[q % 2], outs[j].at[pl.ds(base + c * ch, ch)],
                        sem_w)
                wprev.wait()
                return carry

            lax.fori_loop(0, k_chunks // 2, body, 0)

    flat = [t for t, _ in pairs] + idx_arrays
    return _k(*flat)


def _mm_body(x_ref, w_ref, b_ref, o_ref, *, act):
    y = jnp.dot(x_ref[...], w_ref[...], preferred_element_type=jnp.float32)
    y = y + b_ref[...][None, :]
    if act == "relu":
        y = jnp.maximum(y, 0.0)
    o_ref[...] = y


def _mm(x, w, b=None, act=None, bn=2000):
    n, k = x.shape
    f = w.shape[1]
    if b is None:
        b = jnp.zeros((f,), jnp.float32)
    return pl.pallas_call(
        functools.partial(_mm_body, act=act),
        grid=(n // bn,),
        in_specs=[
            pl.BlockSpec((bn, k), lambda i: (i, 0)),
            pl.BlockSpec((k, f), lambda i: (0, 0)),
            pl.BlockSpec((f,), lambda i: (0,)),
        ],
        out_specs=pl.BlockSpec((bn, f), lambda i: (i, 0)),
        out_shape=jax.ShapeDtypeStruct((n, f), jnp.float32),
    )(x, w, b)


def _gate_body(x_ref, m_ref, wt_ref, wb_ref, gb_ref, den_ref, o_ref, *, relu_m,
               use_den):
    m = m_ref[...]
    if use_den:
        m = m / (den_ref[...] + 1e-16)
    if relu_m:
        m = jnp.maximum(m, 0.0)
    logit = (
        jnp.dot(x_ref[...], wt_ref[...], preferred_element_type=jnp.float32)
        + jnp.dot(m, wb_ref[...], preferred_element_type=jnp.float32)
        + gb_ref[0]
    )
    c = jax.nn.sigmoid(logit)
    o_ref[...] = c * x_ref[...] + (1.0 - c) * m


def _gate(x, m, gw, gb, relu_m, denom=None, bn=2000):
    n, k = x.shape
    wt, wb = gw[:k], gw[k:]
    use_den = denom is not None
    if denom is None:
        denom = jnp.ones((n, 1), jnp.float32)
    return pl.pallas_call(
        functools.partial(_gate_body, relu_m=relu_m, use_den=use_den),
        grid=(n // bn,),
        in_specs=[
            pl.BlockSpec((bn, k), lambda i: (i, 0)),
            pl.BlockSpec((bn, k), lambda i: (i, 0)),
            pl.BlockSpec((k, 1), lambda i: (0, 0)),
            pl.BlockSpec((k, 1), lambda i: (0, 0)),
            pl.BlockSpec((1,), lambda i: (0,)),
            pl.BlockSpec((bn, 1), lambda i: (i, 0)),
        ],
        out_specs=pl.BlockSpec((bn, k), lambda i: (i, 0)),
        out_shape=jax.ShapeDtypeStruct((n, k), jnp.float32),
    )(x, m, wt, wb, gb, denom)


def _pair_body(ga_ref, gb_ref,
               b1e_ref, w2e_ref, b2e_ref, b1d_ref, w2d_ref, b2d_ref,
               rotor_ref, cf_ref, en_ref, dv_ref):
    i = pl.program_id(0)
    nsteps = pl.num_programs(0)

    ga = ga_ref[...]
    gb = gb_ref[...]
    ae, ad, s0 = ga[:, :_DIM], ga[:, _DIM:2 * _DIM], ga[:, 2 * _DIM:]
    be, bd, s1 = gb[:, :_DIM], gb[:, _DIM:2 * _DIM], gb[:, 2 * _DIM:]

    he = jnp.maximum(ae + be + b1e_ref[...][None, :], 0.0)
    eps_logit = jnp.dot(he, w2e_ref[...], preferred_element_type=jnp.float32) + b2e_ref[0]
    eps = jax.nn.sigmoid(eps_logit[:, 0]) * (_ES1 - _ES0) + _ES0

    hd = jnp.maximum(ad + bd + b1d_ref[...][None, :], 0.0)
    dv_logit = jnp.dot(hd, w2d_ref[...], preferred_element_type=jnp.float32) + b2d_ref[0]
    dvdw = jnp.tanh(dv_logit[:, 0]) * _DEV_COEFF
    dx = s0[:, 0] - s1[:, 0]
    dy = s0[:, 1] - s1[:, 1]
    dz = s0[:, 2] - s1[:, 2]
    D = jnp.sqrt(dx * dx + dy * dy + dz * dz + 1e-12)

    lig0, lig1 = s0[:, 4], s1[:, 4]
    met0, met1 = s0[:, 5], s1[:, 5]
    don0, don1 = s0[:, 6], s1[:, 6]
    acc0, acc1 = s0[:, 7], s1[:, 7]
    hyd0, hyd1 = s0[:, 8], s1[:, 8]
    bat0, bat1 = s0[:, 9], s1[:, 9]

    pair_ok = lig0 * (1.0 - lig1) * (bat0 == bat1).astype(jnp.float32)
    maskf = pair_ok * (D >= _IR0).astype(jnp.float32) * (D <= _IR1).astype(jnp.float32)

    R = s0[:, 3] + s1[:, 3] + dvdw
    Dc = jnp.maximum(D, _IR0)
    ratio = R / Dc
    lj = jnp.minimum(ratio ** _N_SHORT - 2.0 * ratio ** _N_LONG, 100.0) * eps

    hbc = cf_ref[0]
    hpc = cf_ref[1]
    rc = cf_ref[2]
    min_hb = -(hbc * hbc)
    min_hp = -(hpc * hpc)
    dev = Dc - R

    def _lp(minima, c0, c1):
        frac = jnp.clip((c1 - dev) / (c1 - c0), 0.0, 1.0)
        return minima * frac

    e_hb = _lp(min_hb, _HB[0], _HB[1])
    e_ml = _lp(min_hb, _ML[0], _ML[1])
    e_hp = _lp(min_hp, _HP[0], _HP[1])

    not_metal = (1.0 - met0) * (1.0 - met1)
    m_hb = jnp.minimum(don0 * acc1 + acc0 * don1, 1.0) * not_metal
    m_ml = jnp.minimum(met0 * acc1 + acc0 * met1, 1.0)
    m_hp = hyd0 * hyd1 * not_metal

    ep = jnp.stack(
        [lj * not_metal, e_hb * m_hb, e_ml * m_ml, e_hp * m_hp], axis=1
    ) * maskf[:, None]

    gi = jax.lax.broadcasted_iota(jnp.int32, (_N_GRAPHS, ep.shape[0]), 0)
    onehot = (gi == bat0.astype(jnp.int32)[None, :]).astype(jnp.float32)
    part = jnp.dot(onehot, ep, preferred_element_type=jnp.float32,
                   precision=jax.lax.Precision.HIGHEST)

    @pl.when(i == 0)
    def _():
        en_ref[...] = jnp.zeros_like(en_ref)

    en_ref[...] += part

    @pl.when(i == nsteps - 1)
    def _():
        penalty = 1.0 + rc * rc * rotor_ref[...]
        en_ref[...] = en_ref[...] / penalty

    dv_ref[...] = (dvdw * maskf)[None, None, :]


def _pair_stage(ga, gb, pe, pd, rotor, coeffs, eb=2000):
    e = ga.shape[0]
    grid = e // eb
    en, dv = pl.pallas_call(
        _pair_body,
        grid=(grid,),
        in_specs=[
            pl.BlockSpec((eb, 3 * _DIM), lambda i: (i, 0)),
            pl.BlockSpec((eb, 3 * _DIM), lambda i: (i, 0)),
            pl.BlockSpec((_DIM,), lambda i: (0,)),
            pl.BlockSpec((_DIM, 1), lambda i: (0, 0)),
            pl.BlockSpec((1,), lambda i: (0,)),
            pl.BlockSpec((_DIM,), lambda i: (0,)),
            pl.BlockSpec((_DIM, 1), lambda i: (0, 0)),
            pl.BlockSpec((1,), lambda i: (0,)),
            pl.BlockSpec((_N_GRAPHS, 1), lambda i: (0, 0)),
            pl.BlockSpec((3,), lambda i: (0,)),
        ],
        out_specs=[
            pl.BlockSpec((_N_GRAPHS, 4), lambda i: (0, 0)),
            pl.BlockSpec((1, 1, eb), lambda i: (i, 0, 0)),
        ],
        out_shape=[
            jax.ShapeDtypeStruct((_N_GRAPHS, 4), jnp.float32),
            jax.ShapeDtypeStruct((grid, 1, eb), jnp.float32),
        ],
    )(ga, gb, pe["b1"], pe["W2"], pe["b2"],
      pd["b1"], pd["W2"], pd["b2"], rotor, coeffs)
    return en, dv.reshape(e)


def kernel(x, edge_index, edge_index_c, edge_index_i, pos, vdw_radii, batch,
           is_ligand, is_metal, is_h_donor, is_h_acceptor, is_hydrophobic,
           rotor, params):
    h = _mm(x, params["embed_W"])

    src, dst = edge_index[0], edge_index[1]
    e_intra = src.shape[0]
    eye = jnp.eye(_DIM, dtype=jnp.float32)
    for p in params["gat"]:
        h1 = _mm(h, p["W"], p["b"])
        pcat = _mm(h1, jnp.concatenate([eye, p["A"]], axis=1))
        ps, pd = _sc_gather_multi([(pcat, src), (pcat, dst)], e_intra,
                                  [2 * _DIM, 2 * _DIM], ch=216)
        ps, pd = ps[:e_intra], pd[:e_intra]
        e = jnp.sum(ps[:, _DIM:] * pd[:, :_DIM]
                    + pd[:, _DIM:] * ps[:, :_DIM], -1)
        w = jnp.exp(jnp.minimum(e, 80.0))
        cat = jnp.concatenate([w[:, None] * ps[:, :_DIM], w[:, None]], axis=1)
        s = jax.ops.segment_sum(cat, dst, num_segments=_N_NODES)
        h = _gate(h, s[:, :_DIM], p["gW"], p["gb"], relu_m=True,
                  denom=s[:, _DIM:])

    srcc, dstc = edge_index_c[0], edge_index_c[1]
    e_inter = srcc.shape[0]
    for p in params["inter"]:
        hr = _mm(h, p["W"], p["b"], act="relu")
        (ghr,) = _sc_gather_multi([(hr, srcc)], e_inter, [_DIM])
        m = jax.ops.segment_sum(ghr[:e_inter], dstc, num_segments=_N_NODES)
        h = _gate(h, m, p["gW"], p["gb"], relu_m=False)

    pe, pdv = params["vdw_eps"], params["dvdw"]

    scal = jnp.concatenate(
        [
            pos,
            vdw_radii[:, None],
            is_ligand[:, None].astype(jnp.float32),
            is_metal[:, None].astype(jnp.float32),
            is_h_donor[:, None].astype(jnp.float32),
            is_h_acceptor[:, None].astype(jnp.float32),
            is_hydrophobic[:, None].astype(jnp.float32),
            batch[:, None].astype(jnp.float32),
            jnp.zeros((_N_NODES, _DIM - 10), jnp.float32),
        ],
        axis=1,
    )

    i0, i1 = edge_index_i[0], edge_index_i[1]
    e_pair = i0.shape[0]
    t0 = jnp.concatenate(
        [_mm(h, jnp.concatenate([pe["W1"][:_DIM], pdv["W1"][:_DIM]], axis=1)),
         scal], axis=1)
    t1 = jnp.concatenate(
        [_mm(h, jnp.concatenate([pe["W1"][_DIM:], pdv["W1"][_DIM:]], axis=1)),
         scal], axis=1)
    ga, gb = _sc_gather_multi([(t0, i0), (t1, i1)], e_pair,
                              [3 * _DIM, 3 * _DIM], ch=128)
    ga, gb = ga[:e_pair], gb[:e_pair]

    coeffs = jnp.concatenate(
        [params["hbond_coeff"], params["hydrophobic_coeff"], params["rotor_coeff"]]
    )
    energies, dvdw_masked = _pair_stage(ga, gb, pe, pdv, rotor, coeffs)
    return energies, dvdw_masked


# speedup vs baseline: 1.1331x; 1.0280x over previous
import functools

import jax
import jax.numpy as jnp
from jax import lax
from jax.experimental import pallas as pl
from jax.experimental.pallas import tpu as pltpu
from jax.experimental.pallas import tpu_sc as plsc

_N_NODES = 10000
_N_GRAPHS = 64
_DIM = 128
_IR0, _IR1 = 0.5, 5.0
_DEV_COEFF = 0.2
_ES0, _ES1 = 0.0178, 0.0356
_N_SHORT, _N_LONG = 10.0, 6.0
_HB = (-0.7, 0.0)
_ML = (-0.7, 0.0)
_HP = (0.5, 1.5)


_NW = 32
_CH = 448


def _sc_gather_multi(pairs, e, out_dims, ch=None):
    ch = ch or _CH
    k_chunks = -(-e // (_NW * ch))
    k_chunks += k_chunks % 2
    e_pad = _NW * ch * k_chunks
    per_w = ch * k_chunks
    mesh = plsc.VectorSubcoreMesh(core_axis_name="c", subcore_axis_name="s")
    n = len(pairs)
    d = out_dims[0]
    assert all(x == d for x in out_dims)

    idx_arrays = []
    idx_slot = []
    for _, ia in pairs:
        for s, a in enumerate(idx_arrays):
            if a is ia:
                idx_slot.append(s)
                break
        else:
            idx_slot.append(len(idx_arrays))
            idx_arrays.append(ia)
    idx_arrays = [jnp.pad(a.astype(jnp.int32), (0, e_pad - e))
                  for a in idx_arrays]
    ni = len(idx_arrays)
    total = k_chunks * n

    @functools.partial(
        pl.kernel,
        mesh=mesh,
        out_type=[jax.ShapeDtypeStruct((e_pad, d) if d else (e_pad,),
                                       jnp.float32)
                  for _ in range(n)],
        scratch_types=[pltpu.VMEM((per_w,), jnp.int32) for _ in range(ni)]
                      + [pltpu.VMEM((ch, d) if d else (ch,), jnp.float32)
                         for _ in range(2)]
                      + [pltpu.SemaphoreType.DMA, pltpu.SemaphoreType.DMA],
    )
    def _k(*refs):
        tables = refs[0:n]
        idxs_hbm = refs[n:n + ni]
        outs = refs[n + ni:2 * n + ni]
        idx_v = refs[2 * n + ni:2 * n + 2 * ni]
        ring = refs[2 * n + 2 * ni:2 * n + 2 * ni + 2]
        sem_g, sem_w = refs[-2:]
        wid = lax.axis_index("s") * 2 + lax.axis_index("c")
        base = wid * per_w
        for s in range(ni):
            pltpu.sync_copy(idxs_hbm[s].at[pl.ds(base, per_w)], idx_v[s])

        if total <= 64:
            def issue(q):
                c, j = divmod(q, n)
                isl = idx_v[idx_slot[j]].at[pl.ds(c * ch, ch)]
                return pltpu.async_copy(tables[j].at[isl], ring[q % 2], sem_g)

            gq = issue(0)
            wprev = None
            for q in range(total):
                gq.wait()
                if wprev is not None:
                    wprev.wait()
                if q + 1 < total:
                    gq = issue(q + 1)
                c, j = divmod(q, n)
                wprev = pltpu.async_copy(
                    ring[q % 2], outs[j].at[pl.ds(base + c * ch, ch)], sem_w)
            wprev.wait()
        else:
            njobs = 2 * n

            def body(ci, carry):
                def issue(q):
                    c = ci * 2 + q // n
                    j = q % n
                    isl = idx_v[idx_slot[j]].at[pl.ds(c * ch, ch)]
                    return pltpu.async_copy(tables[j].at[isl], ring[q % 2],
                                            sem_g)

                gq = issue(0)
                wprev = None
                for q in range(njobs):
                    gq.wait()
                    if wprev is not None:
                        wprev.wait()
                    if q + 1 < njobs:
                        gq = issue(q + 1)
                    c = ci * 2 + q // n
                    j = q % n
                    wprev = pltpu.async_copy(
                        ring[q % 2], outs[j].at[pl.ds(base + c * ch, ch)],
                        sem_w)
                wprev.wait()
                return carry

            lax.fori_loop(0, k_chunks // 2, body, 0)

    flat = [t for t, _ in pairs] + idx_arrays
    return _k(*flat)


def _mm_body(x_ref, w_ref, b_ref, o_ref, *, act):
    y = jnp.dot(x_ref[...], w_ref[...], preferred_element_type=jnp.float32)
    y = y + b_ref[...][None, :]
    if act == "relu":
        y = jnp.maximum(y, 0.0)
    o_ref[...] = y


def _mm(x, w, b=None, act=None, bn=2000):
    n, k = x.shape
    f = w.shape[1]
    if b is None:
        b = jnp.zeros((f,), jnp.float32)
    return pl.pallas_call(
        functools.partial(_mm_body, act=act),
        grid=(n // bn,),
        in_specs=[
            pl.BlockSpec((bn, k), lambda i: (i, 0)),
            pl.BlockSpec((k, f), lambda i: (0, 0)),
            pl.BlockSpec((f,), lambda i: (0,)),
        ],
        out_specs=pl.BlockSpec((bn, f), lambda i: (i, 0)),
        out_shape=jax.ShapeDtypeStruct((n, f), jnp.float32),
    )(x, w, b)


def _gate_body(x_ref, m_ref, wt_ref, wb_ref, gb_ref, den_ref, o_ref, *, relu_m,
               use_den):
    m = m_ref[...]
    if use_den:
        m = m / (den_ref[...] + 1e-16)
    if relu_m:
        m = jnp.maximum(m, 0.0)
    logit = (
        jnp.dot(x_ref[...], wt_ref[...], preferred_element_type=jnp.float32)
        + jnp.dot(m, wb_ref[...], preferred_element_type=jnp.float32)
        + gb_ref[0]
    )
    c = jax.nn.sigmoid(logit)
    o_ref[...] = c * x_ref[...] + (1.0 - c) * m


def _gate(x, m, gw, gb, relu_m, denom=None, bn=2000):
    n, k = x.shape
    wt, wb = gw[:k], gw[k:]
    use_den = denom is not None
    if denom is None:
        denom = jnp.ones((n, 1), jnp.float32)
    return pl.pallas_call(
        functools.partial(_gate_body, relu_m=relu_m, use_den=use_den),
        grid=(n // bn,),
        in_specs=[
            pl.BlockSpec((bn, k), lambda i: (i, 0)),
            pl.BlockSpec((bn, k), lambda i: (i, 0)),
            pl.BlockSpec((k, 1), lambda i: (0, 0)),
            pl.BlockSpec((k, 1), lambda i: (0, 0)),
            pl.BlockSpec((1,), lambda i: (0,)),
            pl.BlockSpec((bn, 1), lambda i: (i, 0)),
        ],
        out_specs=pl.BlockSpec((bn, k), lambda i: (i, 0)),
        out_shape=jax.ShapeDtypeStruct((n, k), jnp.float32),
    )(x, m, wt, wb, gb, denom)


def _pair_body(ga_ref, gb_ref,
               b1e_ref, w2e_ref, b2e_ref, b1d_ref, w2d_ref, b2d_ref,
               rotor_ref, cf_ref, en_ref, dv_ref):
    i = pl.program_id(0)
    nsteps = pl.num_programs(0)

    ga = ga_ref[...]
    gb = gb_ref[...]
    ae, ad, s0 = ga[:, :_DIM], ga[:, _DIM:2 * _DIM], ga[:, 2 * _DIM:]
    be, bd, s1 = gb[:, :_DIM], gb[:, _DIM:2 * _DIM], gb[:, 2 * _DIM:]

    he = jnp.maximum(ae + be + b1e_ref[...][None, :], 0.0)
    eps_logit = jnp.dot(he, w2e_ref[...], preferred_element_type=jnp.float32) + b2e_ref[0]
    eps = jax.nn.sigmoid(eps_logit[:, 0]) * (_ES1 - _ES0) + _ES0

    hd = jnp.maximum(ad + bd + b1d_ref[...][None, :], 0.0)
    dv_logit = jnp.dot(hd, w2d_ref[...], preferred_element_type=jnp.float32) + b2d_ref[0]
    dvdw = jnp.tanh(dv_logit[:, 0]) * _DEV_COEFF
    dx = s0[:, 0] - s1[:, 0]
    dy = s0[:, 1] - s1[:, 1]
    dz = s0[:, 2] - s1[:, 2]
    D = jnp.sqrt(dx * dx + dy * dy + dz * dz + 1e-12)

    lig0, lig1 = s0[:, 4], s1[:, 4]
    met0, met1 = s0[:, 5], s1[:, 5]
    don0, don1 = s0[:, 6], s1[:, 6]
    acc0, acc1 = s0[:, 7], s1[:, 7]
    hyd0, hyd1 = s0[:, 8], s1[:, 8]
    bat0, bat1 = s0[:, 9], s1[:, 9]

    pair_ok = lig0 * (1.0 - lig1) * (bat0 == bat1).astype(jnp.float32)
    maskf = pair_ok * (D >= _IR0).astype(jnp.float32) * (D <= _IR1).astype(jnp.float32)

    R = s0[:, 3] + s1[:, 3] + dvdw
    Dc = jnp.maximum(D, _IR0)
    ratio = R / Dc
    lj = jnp.minimum(ratio ** _N_SHORT - 2.0 * ratio ** _N_LONG, 100.0) * eps

    hbc = cf_ref[0]
    hpc = cf_ref[1]
    rc = cf_ref[2]
    min_hb = -(hbc * hbc)
    min_hp = -(hpc * hpc)
    dev = Dc - R

    def _lp(minima, c0, c1):
        frac = jnp.clip((c1 - dev) / (c1 - c0), 0.0, 1.0)
        return minima * frac

    e_hb = _lp(min_hb, _HB[0], _HB[1])
    e_ml = _lp(min_hb, _ML[0], _ML[1])
    e_hp = _lp(min_hp, _HP[0], _HP[1])

    not_metal = (1.0 - met0) * (1.0 - met1)
    m_hb = jnp.minimum(don0 * acc1 + acc0 * don1, 1.0) * not_metal
    m_ml = jnp.minimum(met0 * acc1 + acc0 * met1, 1.0)
    m_hp = hyd0 * hyd1 * not_metal

    ep = jnp.stack(
        [lj * not_metal, e_hb * m_hb, e_ml * m_ml, e_hp * m_hp], axis=1
    ) * maskf[:, None]

    gi = jax.lax.broadcasted_iota(jnp.int32, (_N_GRAPHS, ep.shape[0]), 0)
    onehot = (gi == bat0.astype(jnp.int32)[None, :]).astype(jnp.float32)
    part = jnp.dot(onehot, ep, preferred_element_type=jnp.float32,
                   precision=jax.lax.Precision.HIGHEST)

    @pl.when(i == 0)
    def _():
        en_ref[...] = jnp.zeros_like(en_ref)

    en_ref[...] += part

    @pl.when(i == nsteps - 1)
    def _():
        penalty = 1.0 + rc * rc * rotor_ref[...]
        en_ref[...] = en_ref[...] / penalty

    dv_ref[...] = (dvdw * maskf)[None, None, :]


def _pair_stage(ga, gb, pe, pd, rotor, coeffs, eb=2000):
    e = ga.shape[0]
    grid = e // eb
    en, dv = pl.pallas_call(
        _pair_body,
        grid=(grid,),
        in_specs=[
            pl.BlockSpec((eb, 3 * _DIM), lambda i: (i, 0)),
            pl.BlockSpec((eb, 3 * _DIM), lambda i: (i, 0)),
            pl.BlockSpec((_DIM,), lambda i: (0,)),
            pl.BlockSpec((_DIM, 1), lambda i: (0, 0)),
            pl.BlockSpec((1,), lambda i: (0,)),
            pl.BlockSpec((_DIM,), lambda i: (0,)),
            pl.BlockSpec((_DIM, 1), lambda i: (0, 0)),
            pl.BlockSpec((1,), lambda i: (0,)),
            pl.BlockSpec((_N_GRAPHS, 1), lambda i: (0, 0)),
            pl.BlockSpec((3,), lambda i: (0,)),
        ],
        out_specs=[
            pl.BlockSpec((_N_GRAPHS, 4), lambda i: (0, 0)),
            pl.BlockSpec((1, 1, eb), lambda i: (i, 0, 0)),
        ],
        out_shape=[
            jax.ShapeDtypeStruct((_N_GRAPHS, 4), jnp.float32),
            jax.ShapeDtypeStruct((grid, 1, eb), jnp.float32),
        ],
    )(ga, gb, pe["b1"], pe["W2"], pe["b2"],
      pd["b1"], pd["W2"], pd["b2"], rotor, coeffs)
    return en, dv.reshape(e)


def kernel(x, edge_index, edge_index_c, edge_index_i, pos, vdw_radii, batch,
           is_ligand, is_metal, is_h_donor, is_h_acceptor, is_hydrophobic,
           rotor, params):
    h = _mm(x, params["embed_W"])

    src, dst = edge_index[0], edge_index[1]
    e_intra = src.shape[0]
    eye = jnp.eye(_DIM, dtype=jnp.float32)
    for p in params["gat"]:
        h1 = _mm(h, p["W"], p["b"])
        pcat = _mm(h1, jnp.concatenate([eye, p["A"]], axis=1))
        ps, pd = _sc_gather_multi([(pcat, src), (pcat, dst)], e_intra,
                                  [2 * _DIM, 2 * _DIM], ch=216)
        ps, pd = ps[:e_intra], pd[:e_intra]
        e = jnp.sum(ps[:, _DIM:] * pd[:, :_DIM]
                    + pd[:, _DIM:] * ps[:, :_DIM], -1)
        w = jnp.exp(jnp.minimum(e, 80.0))
        cat = jnp.concatenate([w[:, None] * ps[:, :_DIM], w[:, None]], axis=1)
        s = jax.ops.segment_sum(cat, dst, num_segments=_N_NODES)
        h = _gate(h, s[:, :_DIM], p["gW"], p["gb"], relu_m=True,
                  denom=s[:, _DIM:])

    srcc, dstc = edge_index_c[0], edge_index_c[1]
    e_inter = srcc.shape[0]
    for p in params["inter"]:
        hr = _mm(h, p["W"], p["b"], act="relu")
        (ghr,) = _sc_gather_multi([(hr, srcc)], e_inter, [_DIM])
        m = jax.ops.segment_sum(ghr[:e_inter], dstc, num_segments=_N_NODES)
        h = _gate(h, m, p["gW"], p["gb"], relu_m=False)

    pe, pdv = params["vdw_eps"], params["dvdw"]

    scal = jnp.concatenate(
        [
            pos,
            vdw_radii[:, None],
            is_ligand[:, None].astype(jnp.float32),
            is_metal[:, None].astype(jnp.float32),
            is_h_donor[:, None].astype(jnp.float32),
            is_h_acceptor[:, None].astype(jnp.float32),
            is_hydrophobic[:, None].astype(jnp.float32),
            batch[:, None].astype(jnp.float32),
            jnp.zeros((_N_NODES, _DIM - 10), jnp.float32),
        ],
        axis=1,
    )

    i0, i1 = edge_index_i[0], edge_index_i[1]
    e_pair = i0.shape[0]
    t0 = jnp.concatenate(
        [_mm(h, jnp.concatenate([pe["W1"][:_DIM], pdv["W1"][:_DIM]], axis=1)),
         scal], axis=1)
    t1 = jnp.concatenate(
        [_mm(h, jnp.concatenate([pe["W1"][_DIM:], pdv["W1"][_DIM:]], axis=1)),
         scal], axis=1)
    ga, gb = _sc_gather_multi([(t0, i0), (t1, i1)], e_pair,
                              [3 * _DIM, 3 * _DIM], ch=144)
    ga, gb = ga[:e_pair], gb[:e_pair]

    coeffs = jnp.concatenate(
        [params["hbond_coeff"], params["hydrophobic_coeff"], params["rotor_coeff"]]
    )
    energies, dvdw_masked = _pair_stage(ga, gb, pe, pdv, rotor, coeffs)
    return energies, dvdw_masked
